# baseline (speedup 1.0000x reference)
import jax
import jax.numpy as jnp
from jax import lax
from jax.experimental import pallas as pl
from jax.experimental.pallas import tpu as pltpu
from jax.experimental.pallas import tpu_sc as plsc

N = 650
NP = 656
E = 41600
HID = 256
BS = 64
NC = 2
NS = 16
GRP = 128
G0 = 20
EPW0 = G0 * GRP
NEXTRA = 5
HROWS = NP // NC
TBLH = HROWS * NP
TBLH_PAD = 215936
CH_Z = TBLH_PAD // NS
CH_O = TBLH // NS


def _sc_count_body(src_hbm, dst_hbm, out_hbm, src_v, dst_v, idx_v, ones_v,
                   zbuf, tbl_sh, sem):
    c = lax.axis_index("c")
    s = lax.axis_index("s")

    base = s * EPW0
    ld1 = pltpu.async_copy(src_hbm.at[pl.ds(base, EPW0)],
                           src_v.at[pl.ds(0, EPW0)], sem)
    ld2 = pltpu.async_copy(dst_hbm.at[pl.ds(base, EPW0)],
                           dst_v.at[pl.ds(0, EPW0)], sem)

    zero16 = jnp.zeros((16,), jnp.float32)

    def _zf(i, carry):
        zbuf[pl.ds(i * 16, 16)] = zero16
        return carry

    lax.fori_loop(0, CH_Z // 16, _zf, 0, unroll=8)

    one16 = jnp.ones((16,), jnp.float32)
    for j in range(GRP // 16):
        ones_v[pl.ds(j * 16, 16)] = one16

    pltpu.sync_copy(zbuf, tbl_sh.at[pl.ds(s * CH_Z, CH_Z)])

    ld1.wait()
    ld2.wait()

    @pl.when(s < NEXTRA)
    def _():
        xb = NS * EPW0 + s * GRP
        pltpu.sync_copy(src_hbm.at[pl.ds(xb, GRP)],
                        src_v.at[pl.ds(EPW0, GRP)])
        pltpu.sync_copy(dst_hbm.at[pl.ds(xb, GRP)],
                        dst_v.at[pl.ds(EPW0, GRP)])

    row0 = c * HROWS
    for g in range(G0 + 1):
        if g == G0:
            do = pl.when(s < NEXTRA)
        else:
            do = pl.when(True)

        @do
        def _(g=g):
            for j in range(GRP // 16):
                o = g * GRP + j * 16
                rel = dst_v[pl.ds(o, 16)] - row0
                idx = rel * NP + src_v[pl.ds(o, 16)]
                ok = (rel >= 0) & (rel < HROWS)
                idx_v[g, pl.ds(j * 16, 16)] = jnp.where(ok, idx, TBLH)

    plsc.subcore_barrier()
    cps = [pltpu.async_copy(ones_v, tbl_sh.at[idx_v.at[g]], sem, add=True)
           for g in range(G0)]

    @pl.when(s < NEXTRA)
    def _():
        pltpu.async_copy(ones_v, tbl_sh.at[idx_v.at[G0]], sem,
                         add=True).wait()

    for cp in cps:
        cp.wait()
    plsc.subcore_barrier()

    off = s * CH_O
    pltpu.sync_copy(tbl_sh.at[pl.ds(off, CH_O)], zbuf.at[pl.ds(0, CH_O)])
    pltpu.sync_copy(zbuf.at[pl.ds(0, CH_O)],
                    out_hbm.at[pl.ds(c * TBLH + off, CH_O)])


_SC_COUNT_CACHE = []


def _sc_count(src, dst):
    if not _SC_COUNT_CACHE:
        _SC_COUNT_CACHE.append(pl.kernel(
            _sc_count_body,
            out_type=jax.ShapeDtypeStruct((NC * TBLH,), jnp.float32),
            mesh=plsc.VectorSubcoreMesh(core_axis_name="c",
                                        subcore_axis_name="s",
                                        num_cores=NC, num_subcores=NS),
            scratch_types=[
                pltpu.VMEM((EPW0 + GRP,), jnp.int32),
                pltpu.VMEM((EPW0 + GRP,), jnp.int32),
                pltpu.VMEM((G0 + 1, GRP), jnp.int32),
                pltpu.VMEM((GRP,), jnp.float32),
                pltpu.VMEM((CH_Z,), jnp.float32),
                pltpu.VMEM_SHARED((TBLH_PAD,), jnp.float32),
                pltpu.SemaphoreType.DMA,
            ],
        ))
    return _SC_COUNT_CACHE[0](src, dst)


def _dense_body(c_ref, x_ref, w1_ref, as1_ref, ad1_ref, b1_ref,
                w4_ref, as4_ref, ad4_ref, b4_ref, out_ref):
    f32 = jnp.float32
    rows = lax.broadcasted_iota(jnp.int32, (NP, NP), 0)
    cols = lax.broadcasted_iota(jnp.int32, (NP, NP), 1)
    C = c_ref[:] + jnp.where(rows == cols, 1.0, 0.0).astype(f32)
    mask = C > 0.0

    def gat(h, att_s, att_d, b):
        a_s = lax.dot_general(att_s, h, (((1,), (1,)), ((), ())),
                              preferred_element_type=f32)
        a_d = jnp.sum(h * att_d, axis=1, keepdims=True)
        alpha = a_d + a_s
        alpha = jnp.where(alpha > 0.0, alpha, 0.2 * alpha)
        am = jnp.max(jnp.where(mask, alpha, -1e30), axis=1, keepdims=True)
        e = jnp.where(mask, C * jnp.exp(alpha - am), 0.0)
        denom = jnp.maximum(jnp.sum(e, axis=1, keepdims=True), 1e-30)
        coef = e / denom
        return jnp.dot(coef, h, preferred_element_type=f32) + b

    h1 = jnp.dot(x_ref[:], w1_ref[:], preferred_element_type=f32)
    h = jnp.maximum(gat(h1, as1_ref[:], ad1_ref[:], b1_ref[:]), 0.0)
    h2 = jnp.dot(h, w4_ref[:], preferred_element_type=f32)
    g = gat(h2, as4_ref[:], ad4_ref[:], b4_ref[:])
    g = jnp.where(g > 0.0, g, 0.01 * g)
    out_ref[:] = g[:N]


def _fc_body(hr_ref, wfc_ref, bfc_ref, out_ref):
    o = jnp.dot(hr_ref[:], wfc_ref[:],
                preferred_element_type=jnp.float32) + bfc_ref[:]
    out_ref[:] = 1.0 / (1.0 + jnp.exp(-o))


def kernel(x_s, x_t, edge_index, edge_attr, batch, W1, att_src1, att_dst1,
           b1, W4, att_src4, att_dst4, b4, Wfc, bfc):
    x = jnp.concatenate([x_s, x_t], axis=0)
    x = jnp.pad(x, ((0, NP - x.shape[0]), (0, 0)))

    tbl = _sc_count(edge_index[0], edge_index[1])
    cmat = tbl.reshape(NP, NP)

    g = pl.pallas_call(
        _dense_body,
        out_shape=jax.ShapeDtypeStruct((N, BS), jnp.float32),
    )(cmat, x, W1,
      att_src1.reshape(1, HID), att_dst1.reshape(1, HID), b1.reshape(1, HID),
      W4, att_src4.reshape(1, BS), att_dst4.reshape(1, BS), b4.reshape(1, BS))

    hr = g.reshape(BS, N)
    out = pl.pallas_call(
        _fc_body,
        out_shape=jax.ShapeDtypeStruct((BS, 1), jnp.float32),
    )(hr, Wfc, bfc.reshape(1, 1))
    return out.reshape(1, BS)


# speedup vs baseline: 1.8065x; 1.8065x over previous
import jax
import jax.numpy as jnp
from jax import lax
from jax.experimental import pallas as pl
from jax.experimental.pallas import tpu as pltpu
from jax.experimental.pallas import tpu_sc as plsc

N = 650
NP = 768
PL = 6
E = 41600
HID = 256
BS = 64
NC = 2
NS = 16
GRP = 128
G0 = 20
EPW0 = G0 * GRP
NEXTRA = 5
HROWS = NP // NC
PLW = HROWS * GRP
TBLH = PL * PLW
SCRAP = 1024
TBLH_PAD = TBLH + SCRAP
CH_Z = TBLH_PAD // NS
CH_P = PLW // NS


def _sc_count_body(edge_hbm, out_hbm, src_v, dst_v, idx_v, ones_v,
                   zbuf, tbl_sh, sem):
    c = lax.axis_index("c")
    s = lax.axis_index("s")

    base = s * EPW0
    ld1 = pltpu.async_copy(edge_hbm.at[0, pl.ds(base, EPW0)],
                           src_v.at[pl.ds(0, EPW0)], sem)
    ld2 = pltpu.async_copy(edge_hbm.at[1, pl.ds(base, EPW0)],
                           dst_v.at[pl.ds(0, EPW0)], sem)

    zero16 = jnp.zeros((16,), jnp.float32)

    def _zf(i, carry):
        zbuf[pl.ds(i * 16, 16)] = zero16
        return carry

    lax.fori_loop(0, CH_Z // 16, _zf, 0, unroll=8)

    one16 = jnp.ones((16,), jnp.float32)
    for j in range(GRP // 16):
        ones_v[pl.ds(j * 16, 16)] = one16

    pltpu.sync_copy(zbuf, tbl_sh.at[pl.ds(s * CH_Z, CH_Z)])

    ld1.wait()
    ld2.wait()

    @pl.when(s < NEXTRA)
    def _():
        xb = NS * EPW0 + s * GRP
        pltpu.sync_copy(edge_hbm.at[0, pl.ds(xb, GRP)],
                        src_v.at[pl.ds(EPW0, GRP)])
        pltpu.sync_copy(edge_hbm.at[1, pl.ds(xb, GRP)],
                        dst_v.at[pl.ds(EPW0, GRP)])

    row0 = c * HROWS
    lane = lax.iota(jnp.int32, 16)

    def _compute_group(g):
        for j in range(GRP // 16):
            o = g * GRP + j * 16
            sv = src_v[pl.ds(o, 16)]
            dv = dst_v[pl.ds(o, 16)]
            rel = dv - row0
            idx = (sv >> 7) * PLW + rel * GRP + (sv & 127)
            ok = (rel >= 0) & (rel < HROWS)
            scrap = TBLH + ((o + lane) & (SCRAP - 1))
            idx_v[g, pl.ds(j * 16, 16)] = jnp.where(ok, idx, scrap)

    for g in range(G0):
        _compute_group(g)

    @pl.when(s < NEXTRA)
    def _():
        _compute_group(G0)

    plsc.subcore_barrier()
    cps = [pltpu.async_copy(ones_v, tbl_sh.at[idx_v.at[g]], sem, add=True)
           for g in range(G0)]

    @pl.when(s < NEXTRA)
    def _():
        pltpu.async_copy(ones_v, tbl_sh.at[idx_v.at[G0]], sem,
                         add=True).wait()

    for cp in cps:
        cp.wait()
    plsc.subcore_barrier()

    ins = [pltpu.async_copy(tbl_sh.at[pl.ds(p * PLW + s * CH_P, CH_P)],
                            zbuf.at[pl.ds(p * CH_P, CH_P)], sem)
           for p in range(PL)]
    for cp in ins:
        cp.wait()
    outs = [pltpu.async_copy(
                zbuf.at[pl.ds(p * CH_P, CH_P)],
                out_hbm.at[pl.ds(p * NC * PLW + c * PLW + s * CH_P, CH_P)],
                sem)
            for p in range(PL)]
    for cp in outs:
        cp.wait()


_SC_COUNT_CACHE = []


def _sc_count(edge_index):
    if not _SC_COUNT_CACHE:
        _SC_COUNT_CACHE.append(pl.kernel(
            _sc_count_body,
            out_type=jax.ShapeDtypeStruct((PL * NP * GRP,), jnp.float32),
            mesh=plsc.VectorSubcoreMesh(core_axis_name="c",
                                        subcore_axis_name="s",
                                        num_cores=NC, num_subcores=NS),
            scratch_types=[
                pltpu.VMEM((EPW0 + GRP,), jnp.int32),
                pltpu.VMEM((EPW0 + GRP,), jnp.int32),
                pltpu.VMEM((G0 + 1, GRP), jnp.int32),
                pltpu.VMEM((GRP,), jnp.float32),
                pltpu.VMEM((CH_Z,), jnp.float32),
                pltpu.VMEM_SHARED((TBLH_PAD,), jnp.float32),
                pltpu.SemaphoreType.DMA,
            ],
        ))
    return _SC_COUNT_CACHE[0](edge_index)


def _dense_body(c_ref, x_ref, w1_ref, as1_ref, ad1_ref, b1_ref,
                w4_ref, as4_ref, ad4_ref, b4_ref, p_ref, bfc_ref, out_ref):
    f32 = jnp.float32
    rowsb = lax.broadcasted_iota(jnp.int32, (NP, GRP), 0)
    colsb = lax.broadcasted_iota(jnp.int32, (NP, GRP), 1)
    Cs, masks = [], []
    for k in range(PL):
        Ck = c_ref[k] + jnp.where(rowsb == colsb + k * GRP, 1.0, 0.0)
        Cs.append(Ck)
        masks.append(Ck > 0.0)

    def gat(h, att_s, att_d, b):
        a_s = lax.dot_general(att_s, h, (((1,), (1,)), ((), ())),
                              preferred_element_type=f32)
        a_d = jnp.sum(h * att_d, axis=1, keepdims=True)
        alphas, ams = [], []
        for k in range(PL):
            al = a_d + a_s[:, k * GRP:(k + 1) * GRP]
            al = jnp.where(al > 0.0, al, 0.2 * al)
            alphas.append(al)
            ams.append(jnp.max(jnp.where(masks[k], al, -1e30), axis=1,
                               keepdims=True))
        am = ams[0]
        for k in range(1, PL):
            am = jnp.maximum(am, ams[k])
        es = [jnp.where(masks[k], Cs[k] * jnp.exp(alphas[k] - am), 0.0)
              for k in range(PL)]
        denom = es[0].sum(axis=1, keepdims=True)
        for k in range(1, PL):
            denom = denom + es[k].sum(axis=1, keepdims=True)
        denom = jnp.maximum(denom, 1e-30)
        acc = b
        for k in range(PL):
            acc = acc + jnp.dot(es[k] / denom, h[k * GRP:(k + 1) * GRP, :],
                                preferred_element_type=f32)
        return acc

    h1 = jnp.dot(x_ref[:], w1_ref[:], preferred_element_type=f32)
    h = jnp.maximum(gat(h1, as1_ref[:], ad1_ref[:], b1_ref[:]), 0.0)
    h2 = jnp.dot(h, w4_ref[:], preferred_element_type=f32)
    g = gat(h2, as4_ref[:], ad4_ref[:], b4_ref[:])
    g = jnp.where(g > 0.0, g, 0.01 * g)

    contrib = g[:N] * p_ref[:]
    r_i = lax.broadcasted_iota(jnp.int32, (N, BS), 0)
    c_i = lax.broadcasted_iota(jnp.int32, (N, BS), 1)
    f_i = r_i * BS + c_i
    in_first = (f_i // N) == ((r_i * BS) // N)
    s0 = jnp.sum(jnp.where(in_first, contrib, 0.0), axis=1, keepdims=True)
    s1 = jnp.sum(jnp.where(in_first, 0.0, contrib), axis=1, keepdims=True)
    bb = lax.broadcasted_iota(jnp.int32, (BS, N), 0)
    rr = lax.broadcasted_iota(jnp.int32, (BS, N), 1)
    b0 = (rr * BS) // N
    oh0 = jnp.where(b0 == bb, 1.0, 0.0)
    oh1 = jnp.where(b0 + 1 == bb, 1.0, 0.0)
    dn = (((0,), (1,)), ((), ()))
    o = (lax.dot_general(s0, oh0, dn, preferred_element_type=f32)
         + lax.dot_general(s1, oh1, dn, preferred_element_type=f32)
         + bfc_ref[:])
    out_ref[:] = 1.0 / (1.0 + jnp.exp(-o))


def kernel(x_s, x_t, edge_index, edge_attr, batch, W1, att_src1, att_dst1,
           b1, W4, att_src4, att_dst4, b4, Wfc, bfc):
    x = jnp.concatenate([x_s, x_t], axis=0)
    x = jnp.pad(x, ((0, NP - x.shape[0]), (0, 0)))

    tbl = _sc_count(edge_index)
    cpl = tbl.reshape(PL, NP, GRP)

    P = jnp.tile(Wfc[:, 0], BS)[:N * BS].reshape(N, BS)

    out = pl.pallas_call(
        _dense_body,
        out_shape=jax.ShapeDtypeStruct((1, BS), jnp.float32),
    )(cpl, x, W1,
      att_src1.reshape(1, HID), att_dst1.reshape(1, HID), b1.reshape(1, HID),
      W4, att_src4.reshape(1, BS), att_dst4.reshape(1, BS), b4.reshape(1, BS),
      P, bfc.reshape(1, 1))
    return out


# speedup vs baseline: 1.8483x; 1.0231x over previous
import jax
import jax.numpy as jnp
from jax import lax
from jax.experimental import pallas as pl
from jax.experimental.pallas import tpu as pltpu
from jax.experimental.pallas import tpu_sc as plsc

N = 650
NP = 768
PL = 6
E = 41600
HID = 256
BS = 64
NC = 2
NS = 16
GRP = 128
G0 = 20
EPW0 = G0 * GRP
NEXTRA = 5
HROWS = NP // NC
PLW = HROWS * GRP
TBLH = PL * PLW
SCRAP = 1024
TBLH_PAD = TBLH + SCRAP
CH_Z = TBLH_PAD // NS
CH_P = PLW // NS


def _sc_count_body(edge_hbm, out_hbm, src_v, dst_v, idx_v, ones_v,
                   zbuf, tbl_sh, sem):
    c = lax.axis_index("c")
    s = lax.axis_index("s")

    base = s * EPW0
    ld1 = pltpu.async_copy(edge_hbm.at[0, pl.ds(base, EPW0)],
                           src_v.at[pl.ds(0, EPW0)], sem)
    ld2 = pltpu.async_copy(edge_hbm.at[1, pl.ds(base, EPW0)],
                           dst_v.at[pl.ds(0, EPW0)], sem)

    zero16 = jnp.zeros((16,), jnp.float32)

    def _zf(i, carry):
        zbuf[pl.ds(i * 16, 16)] = zero16
        return carry

    lax.fori_loop(0, CH_Z // 16, _zf, 0, unroll=8)

    one16 = jnp.ones((16,), jnp.float32)
    for j in range(GRP // 16):
        ones_v[pl.ds(j * 16, 16)] = one16

    pltpu.sync_copy(zbuf, tbl_sh.at[pl.ds(s * CH_Z, CH_Z)])

    ld1.wait()
    ld2.wait()

    @pl.when(s < NEXTRA)
    def _():
        xb = NS * EPW0 + s * GRP
        pltpu.sync_copy(edge_hbm.at[0, pl.ds(xb, GRP)],
                        src_v.at[pl.ds(EPW0, GRP)])
        pltpu.sync_copy(edge_hbm.at[1, pl.ds(xb, GRP)],
                        dst_v.at[pl.ds(EPW0, GRP)])

    row0 = c * HROWS
    lane = lax.iota(jnp.int32, 16)
    n_g = jnp.where(s < NEXTRA, G0 + 1, G0)

    def _idx_body(g, carry):
        for j in range(GRP // 16):
            o = g * GRP + j * 16
            sv = src_v[pl.ds(o, 16)]
            dv = dst_v[pl.ds(o, 16)]
            rel = dv - row0
            idx = (sv >> 7) * PLW + rel * GRP + (sv & 127)
            ok = (rel >= 0) & (rel < HROWS)
            scrap = TBLH + ((o + lane) & (SCRAP - 1))
            idx_v[g, pl.ds(j * 16, 16)] = jnp.where(ok, idx, scrap)
        return carry

    lax.fori_loop(0, n_g, _idx_body, 0)

    plsc.subcore_barrier()

    def _fire(g, carry):
        pltpu.async_copy(ones_v, tbl_sh.at[idx_v.at[g]], sem, add=True)
        return carry

    lax.fori_loop(0, G0, _fire, 0)

    @pl.when(s < NEXTRA)
    def _():
        pltpu.sync_copy(ones_v, tbl_sh.at[idx_v.at[G0]], add=True)

    pltpu.make_async_copy(out_hbm.at[pl.ds(0, EPW0)],
                          zbuf.at[pl.ds(0, EPW0)], sem).wait()
    plsc.subcore_barrier()

    def _pull(p, carry):
        pltpu.async_copy(tbl_sh.at[pl.ds(p * PLW + s * CH_P, CH_P)],
                         zbuf.at[pl.ds(p * CH_P, CH_P)], sem)
        return carry

    lax.fori_loop(0, PL, _pull, 0)
    pltpu.make_async_copy(out_hbm.at[pl.ds(0, PL * CH_P)],
                          zbuf.at[pl.ds(0, PL * CH_P)], sem).wait()

    def _push(p, carry):
        pltpu.async_copy(
            zbuf.at[pl.ds(p * CH_P, CH_P)],
            out_hbm.at[pl.ds(p * NC * PLW + c * PLW + s * CH_P, CH_P)],
            sem)
        return carry

    lax.fori_loop(0, PL, _push, 0)
    pltpu.make_async_copy(out_hbm.at[pl.ds(0, PL * CH_P)],
                          zbuf.at[pl.ds(0, PL * CH_P)], sem).wait()


_SC_COUNT_CACHE = []


def _sc_count(edge_index):
    if not _SC_COUNT_CACHE:
        _SC_COUNT_CACHE.append(pl.kernel(
            _sc_count_body,
            out_type=jax.ShapeDtypeStruct((PL * NP * GRP,), jnp.float32),
            mesh=plsc.VectorSubcoreMesh(core_axis_name="c",
                                        subcore_axis_name="s",
                                        num_cores=NC, num_subcores=NS),
            scratch_types=[
                pltpu.VMEM((EPW0 + GRP,), jnp.int32),
                pltpu.VMEM((EPW0 + GRP,), jnp.int32),
                pltpu.VMEM((G0 + 1, GRP), jnp.int32),
                pltpu.VMEM((GRP,), jnp.float32),
                pltpu.VMEM((CH_Z,), jnp.float32),
                pltpu.VMEM_SHARED((TBLH_PAD,), jnp.float32),
                pltpu.SemaphoreType.DMA,
            ],
        ))
    return _SC_COUNT_CACHE[0](edge_index)


def _dense_body(c_ref, x_ref, w1_ref, as1_ref, ad1_ref, b1_ref,
                w4_ref, as4_ref, ad4_ref, b4_ref, p_ref, bfc_ref, out_ref):
    f32 = jnp.float32
    rowsb = lax.broadcasted_iota(jnp.int32, (NP, GRP), 0)
    colsb = lax.broadcasted_iota(jnp.int32, (NP, GRP), 1)
    Cs, masks = [], []
    for k in range(PL):
        Ck = c_ref[k] + jnp.where(rowsb == colsb + k * GRP, 1.0, 0.0)
        Cs.append(Ck)
        masks.append(Ck > 0.0)

    def gat(h, att_s, att_d, b):
        a_s = lax.dot_general(att_s, h, (((1,), (1,)), ((), ())),
                              preferred_element_type=f32)
        a_d = jnp.sum(h * att_d, axis=1, keepdims=True)
        alphas, ams = [], []
        for k in range(PL):
            al = a_d + a_s[:, k * GRP:(k + 1) * GRP]
            al = jnp.where(al > 0.0, al, 0.2 * al)
            alphas.append(al)
            ams.append(jnp.max(jnp.where(masks[k], al, -1e30), axis=1,
                               keepdims=True))
        am = ams[0]
        for k in range(1, PL):
            am = jnp.maximum(am, ams[k])
        es = [jnp.where(masks[k], Cs[k] * jnp.exp(alphas[k] - am), 0.0)
              for k in range(PL)]
        denom = es[0].sum(axis=1, keepdims=True)
        for k in range(1, PL):
            denom = denom + es[k].sum(axis=1, keepdims=True)
        denom = jnp.maximum(denom, 1e-30)
        acc = b
        for k in range(PL):
            acc = acc + jnp.dot(es[k] / denom, h[k * GRP:(k + 1) * GRP, :],
                                preferred_element_type=f32)
        return acc

    h1 = jnp.dot(x_ref[:], w1_ref[:], preferred_element_type=f32)
    h = jnp.maximum(gat(h1, as1_ref[:], ad1_ref[:], b1_ref[:]), 0.0)
    h2 = jnp.dot(h, w4_ref[:], preferred_element_type=f32)
    g = gat(h2, as4_ref[:], ad4_ref[:], b4_ref[:])
    g = jnp.where(g > 0.0, g, 0.01 * g)

    contrib = g[:N] * p_ref[:]
    r_i = lax.broadcasted_iota(jnp.int32, (N, BS), 0)
    c_i = lax.broadcasted_iota(jnp.int32, (N, BS), 1)
    f_i = r_i * BS + c_i
    in_first = (f_i // N) == ((r_i * BS) // N)
    s0 = jnp.sum(jnp.where(in_first, contrib, 0.0), axis=1, keepdims=True)
    s1 = jnp.sum(jnp.where(in_first, 0.0, contrib), axis=1, keepdims=True)
    bb = lax.broadcasted_iota(jnp.int32, (BS, N), 0)
    rr = lax.broadcasted_iota(jnp.int32, (BS, N), 1)
    b0 = (rr * BS) // N
    oh0 = jnp.where(b0 == bb, 1.0, 0.0)
    oh1 = jnp.where(b0 + 1 == bb, 1.0, 0.0)
    dn = (((0,), (1,)), ((), ()))
    o = (lax.dot_general(s0, oh0, dn, preferred_element_type=f32)
         + lax.dot_general(s1, oh1, dn, preferred_element_type=f32)
         + bfc_ref[:])
    out_ref[:] = 1.0 / (1.0 + jnp.exp(-o))


def kernel(x_s, x_t, edge_index, edge_attr, batch, W1, att_src1, att_dst1,
           b1, W4, att_src4, att_dst4, b4, Wfc, bfc):
    x = jnp.concatenate([x_s, x_t], axis=0)
    x = jnp.pad(x, ((0, NP - x.shape[0]), (0, 0)))

    tbl = _sc_count(edge_index)
    cpl = tbl.reshape(PL, NP, GRP)

    P = jnp.tile(Wfc[:, 0], BS)[:N * BS].reshape(N, BS)

    out = pl.pallas_call(
        _dense_body,
        out_shape=jax.ShapeDtypeStruct((1, BS), jnp.float32),
    )(cpl, x, W1,
      att_src1.reshape(1, HID), att_dst1.reshape(1, HID), b1.reshape(1, HID),
      W4, att_src4.reshape(1, BS), att_dst4.reshape(1, BS), b4.reshape(1, BS),
      P, bfc.reshape(1, 1))
    return out
